# Initial kernel scaffold; baseline (speedup 1.0000x reference)
import jax
import jax.numpy as jnp
from jax.experimental import pallas as pl


def kernel(input_ids, token_type_ids, word_emb, pos_emb, type_emb, ln_w, ln_b):
    raise NotImplementedError("write your pallas kernel here")


# speedup vs baseline: 8.1671x; 8.1671x over previous
import functools

import jax
import jax.numpy as jnp
from jax import lax
from jax.experimental import pallas as pl
from jax.experimental.pallas import tpu as pltpu
from jax.experimental.pallas import tpu_sc as plsc

_NC = 2
_NS = 16
_NW = _NC * _NS
_GC = 128
_NBUF = 2


def _sc_gather(table, idx):
    n = idx.shape[0]
    d = table.shape[1]
    per_w = n // _NW
    steps = per_w // _GC
    supersteps = steps // _NBUF
    assert per_w * _NW == n and steps * _GC == per_w and supersteps * _NBUF == steps

    mesh = plsc.VectorSubcoreMesh(
        core_axis_name="c", subcore_axis_name="s", num_cores=_NC, num_subcores=_NS
    )

    @functools.partial(
        pl.kernel,
        out_type=jax.ShapeDtypeStruct((n, d), jnp.float32),
        mesh=mesh,
        scratch_types=[
            pltpu.VMEM((_NBUF, _GC), jnp.int32),
            pltpu.VMEM((_NBUF, _GC, d), jnp.float32),
            pltpu.SemaphoreType.DMA,
        ],
    )
    def k(table_hbm, idx_hbm, out_hbm, idx_v, rows_v, gsem):
        wid = lax.axis_index("s") * _NC + lax.axis_index("c")
        base = wid * per_w
        for b in range(_NBUF):
            pltpu.sync_copy(idx_hbm.at[pl.ds(base + b * _GC, _GC)], idx_v.at[b])
            pltpu.async_copy(table_hbm.at[idx_v.at[b]], rows_v.at[b], gsem)

        def body(s, carry):
            off = s * (_NBUF * _GC)
            for b in range(_NBUF):
                row0 = off + b * _GC
                pltpu.make_async_copy(
                    table_hbm.at[idx_v.at[b]], rows_v.at[b], gsem
                ).wait()
                pltpu.sync_copy(rows_v.at[b], out_hbm.at[pl.ds(base + row0, _GC)])
                nxt = row0 + _NBUF * _GC

                @pl.when(nxt < per_w)
                def _():
                    pltpu.sync_copy(idx_hbm.at[pl.ds(base + nxt, _GC)], idx_v.at[b])
                    pltpu.async_copy(table_hbm.at[idx_v.at[b]], rows_v.at[b], gsem)

            return carry

        lax.fori_loop(0, supersteps, body, 0)

    return k(table, idx)


def _ln_body(x_ref, tt_ref, pos_ref, ty_ref, w_ref, b_ref, o_ref):
    x = x_ref[...]
    t = tt_ref[...][..., None]
    ty = ty_ref[...]
    e = x + pos_ref[...][None] + ty[0] + t * (ty[1] - ty[0])
    u = jnp.mean(e, axis=-1, keepdims=True)
    c = e - u
    v = jnp.mean(c * c, axis=-1, keepdims=True)
    o_ref[...] = c * lax.rsqrt(v + 1e-12) * w_ref[...] + b_ref[...]


def _tc_ln(x, tt, pos, ty, w, b):
    bsz, l, h = x.shape
    bb = 64
    return pl.pallas_call(
        _ln_body,
        grid=(bsz // bb,),
        in_specs=[
            pl.BlockSpec((bb, l, h), lambda i: (i, 0, 0)),
            pl.BlockSpec((bb, l), lambda i: (i, 0)),
            pl.BlockSpec((l, h), lambda i: (0, 0)),
            pl.BlockSpec((2, h), lambda i: (0, 0)),
            pl.BlockSpec((1, h), lambda i: (0, 0)),
            pl.BlockSpec((1, h), lambda i: (0, 0)),
        ],
        out_specs=pl.BlockSpec((bb, l, h), lambda i: (i, 0, 0)),
        out_shape=jax.ShapeDtypeStruct((bsz, l, h), jnp.float32),
    )(x, tt, pos, ty, w, b)


def kernel(input_ids, token_type_ids, word_emb, pos_emb, type_emb, ln_w, ln_b):
    bsz, l = input_ids.shape
    h = word_emb.shape[1]
    ids = input_ids.reshape(-1).astype(jnp.int32)
    words = _sc_gather(word_emb, ids)
    tt = token_type_ids.astype(jnp.float32)
    return _tc_ln(
        words.reshape(bsz, l, h),
        tt,
        pos_emb[:l],
        type_emb,
        ln_w.reshape(1, h),
        ln_b.reshape(1, h),
    )


# speedup vs baseline: 8.3001x; 1.0163x over previous
import functools

import jax
import jax.numpy as jnp
from jax import lax
from jax.experimental import pallas as pl
from jax.experimental.pallas import tpu as pltpu
from jax.experimental.pallas import tpu_sc as plsc

_NC = 2
_NS = 16
_NW = _NC * _NS
_GC = 128
_NBUF = 5
_LOOK = 3


def _sc_gather(table, idx3):
    nw, steps, gc = idx3.shape
    d = table.shape[1]
    per_w = steps * gc
    n = nw * per_w
    supersteps = steps // _NBUF
    assert nw == _NW and gc == _GC and supersteps * _NBUF == steps

    mesh = plsc.VectorSubcoreMesh(
        core_axis_name="c", subcore_axis_name="s", num_cores=_NC, num_subcores=_NS
    )

    @functools.partial(
        pl.kernel,
        out_type=jax.ShapeDtypeStruct((n, d), jnp.float32),
        mesh=mesh,
        scratch_types=[
            pltpu.VMEM((steps, _GC), jnp.int32),
            pltpu.VMEM((_NBUF, _GC, d), jnp.float32),
            pltpu.SemaphoreType.DMA((_NBUF,)),
            pltpu.SemaphoreType.DMA((_NBUF,)),
        ],
    )
    def k(table_hbm, idx_hbm, out_hbm, idx_v, rows_v, gsem, ssem):
        wid = lax.axis_index("s") * _NC + lax.axis_index("c")
        base = wid * per_w
        pltpu.sync_copy(idx_hbm.at[wid], idx_v)
        for b in range(_LOOK):
            pltpu.async_copy(table_hbm.at[idx_v.at[b]], rows_v.at[b], gsem.at[b])

        def body(s, carry):
            for b in range(_NBUF):
                g = s * _NBUF + b
                pltpu.make_async_copy(
                    table_hbm.at[idx_v.at[g]], rows_v.at[b], gsem.at[b]
                ).wait()
                pltpu.async_copy(
                    rows_v.at[b], out_hbm.at[pl.ds(base + g * _GC, _GC)], ssem.at[b]
                )
                gn = g + _LOOK
                bn = (b + _LOOK) % _NBUF

                @pl.when(jnp.logical_and(gn < steps, g >= _NBUF - _LOOK))
                def _():
                    pltpu.make_async_copy(
                        rows_v.at[bn],
                        out_hbm.at[pl.ds(base + (gn - _NBUF) * _GC, _GC)],
                        ssem.at[bn],
                    ).wait()

                @pl.when(gn < steps)
                def _():
                    pltpu.async_copy(
                        table_hbm.at[idx_v.at[gn]], rows_v.at[bn], gsem.at[bn]
                    )

            return carry

        lax.fori_loop(0, supersteps, body, 0)
        for b in range(_NBUF):
            pltpu.make_async_copy(
                rows_v.at[b], out_hbm.at[pl.ds(base, _GC)], ssem.at[b]
            ).wait()

    return k(table, idx3)


def _ln_body(x_ref, tt_ref, pos_ref, ty_ref, w_ref, b_ref, o_ref):
    bb, l, h = x_ref.shape
    x = x_ref[...]
    t = tt_ref[...][..., None]
    ty = ty_ref[...]
    e = (x + pos_ref[...][None] + ty[0] + t * (ty[1] - ty[0])).reshape(bb * l, h)
    ones = jnp.ones((h, 1), jnp.float32)
    s1 = jnp.dot(e, ones, preferred_element_type=jnp.float32)
    s2 = jnp.dot(e * e, ones, preferred_element_type=jnp.float32)
    u = s1 * (1.0 / h)
    v = s2 * (1.0 / h) - u * u
    r = lax.rsqrt(v + 1e-12)
    out = (e - u) * (r * w_ref[...]) + b_ref[...]
    o_ref[...] = out.reshape(bb, l, h)


def _tc_ln(x, tt, pos, ty, w, b):
    bsz, l, h = x.shape
    bb = 64
    return pl.pallas_call(
        _ln_body,
        grid=(bsz // bb,),
        in_specs=[
            pl.BlockSpec((bb, l, h), lambda i: (i, 0, 0)),
            pl.BlockSpec((bb, l), lambda i: (i, 0)),
            pl.BlockSpec((l, h), lambda i: (0, 0)),
            pl.BlockSpec((2, h), lambda i: (0, 0)),
            pl.BlockSpec((1, h), lambda i: (0, 0)),
            pl.BlockSpec((1, h), lambda i: (0, 0)),
        ],
        out_specs=pl.BlockSpec((bb, l, h), lambda i: (i, 0, 0)),
        out_shape=jax.ShapeDtypeStruct((bsz, l, h), jnp.float32),
    )(x, tt, pos, ty, w, b)


def kernel(input_ids, token_type_ids, word_emb, pos_emb, type_emb, ln_w, ln_b):
    bsz, l = input_ids.shape
    h = word_emb.shape[1]
    ids = input_ids.reshape(_NW, -1, _GC).astype(jnp.int32)
    words = _sc_gather(word_emb, ids)
    tt = token_type_ids.astype(jnp.float32)
    return _tc_ln(
        words.reshape(bsz, l, h),
        tt,
        pos_emb[:l],
        type_emb,
        ln_w.reshape(1, h),
        ln_b.reshape(1, h),
    )


# speedup vs baseline: 9.4483x; 1.1383x over previous
import functools

import jax
import jax.numpy as jnp
from jax import lax
from jax.experimental import pallas as pl
from jax.experimental.pallas import tpu as pltpu
from jax.experimental.pallas import tpu_sc as plsc

_NC = 2
_NS = 16
_NW = _NC * _NS
_GC = 80
_NBUF = 4
_LOOK = 2
_NCH = 4


def _sc_gather(table, idx3):
    nw, steps, gc = idx3.shape
    d = table.shape[1]
    per_w = steps * gc
    n = nw * per_w
    supersteps = steps // _NBUF
    assert nw == _NW and gc == _GC and supersteps * _NBUF == steps

    mesh = plsc.VectorSubcoreMesh(
        core_axis_name="c", subcore_axis_name="s", num_cores=_NC, num_subcores=_NS
    )

    @functools.partial(
        pl.kernel,
        out_type=jax.ShapeDtypeStruct((n, d), jnp.float32),
        mesh=mesh,
        scratch_types=[
            pltpu.VMEM((steps, _GC), jnp.int32),
            pltpu.VMEM((_NBUF, _GC, d), jnp.float32),
            pltpu.SemaphoreType.DMA((_NBUF,)),
            pltpu.SemaphoreType.DMA((_NBUF,)),
        ],
    )
    def k(table_hbm, idx_hbm, out_hbm, idx_v, rows_v, gsem, ssem):
        wid = lax.axis_index("s") * _NC + lax.axis_index("c")
        base = wid * per_w
        pltpu.sync_copy(idx_hbm.at[wid], idx_v)
        for b in range(_LOOK):
            pltpu.async_copy(table_hbm.at[idx_v.at[b]], rows_v.at[b], gsem.at[b])

        def body(s, carry):
            for b in range(_NBUF):
                g = s * _NBUF + b
                pltpu.make_async_copy(
                    table_hbm.at[idx_v.at[g]], rows_v.at[b], gsem.at[b]
                ).wait()
                pltpu.async_copy(
                    rows_v.at[b], out_hbm.at[pl.ds(base + g * _GC, _GC)], ssem.at[b]
                )
                gn = g + _LOOK
                bn = (b + _LOOK) % _NBUF

                @pl.when(jnp.logical_and(gn < steps, g >= _NBUF - _LOOK))
                def _():
                    pltpu.make_async_copy(
                        rows_v.at[bn],
                        out_hbm.at[pl.ds(base + (gn - _NBUF) * _GC, _GC)],
                        ssem.at[bn],
                    ).wait()

                @pl.when(gn < steps)
                def _():
                    pltpu.async_copy(
                        table_hbm.at[idx_v.at[gn]], rows_v.at[bn], gsem.at[bn]
                    )

            return carry

        lax.fori_loop(0, supersteps, body, 0)
        for b in range(_NBUF):
            pltpu.make_async_copy(
                rows_v.at[b], out_hbm.at[pl.ds(base, _GC)], ssem.at[b]
            ).wait()

    return k(table, idx3)


def _ln_body(prev_ref, x_ref, tt_ref, pc_ref, dty_ref, w_ref, b_ref, o_ref):
    bb, l, h = x_ref.shape
    x = x_ref[...]
    t = tt_ref[...][..., None]
    e = (x + pc_ref[...][None] + t * dty_ref[...]).reshape(bb * l, h)
    oh = jnp.full((h, h), 1.0 / h, jnp.float32)
    u = jnp.dot(e, oh, preferred_element_type=jnp.float32)
    m2 = jnp.dot(e * e, oh, preferred_element_type=jnp.float32)
    r = lax.rsqrt(m2 - u * u + 1e-12)
    out = (e - u) * (r * w_ref[...]) + b_ref[...]
    o_ref[...] = out.reshape(bb, l, h)


def _tc_ln(prev, c, bsz, x, tt, pc, dty, w, b):
    bc, l, h = x.shape
    bb = 64
    nblk = bc // bb
    specs = [
        pl.BlockSpec((bb, l, h), lambda i: (i, 0, 0)),
        pl.BlockSpec((bb, l), lambda i: (i, 0)),
        pl.BlockSpec((l, h), lambda i: (0, 0)),
        pl.BlockSpec((1, h), lambda i: (0, 0)),
        pl.BlockSpec((1, h), lambda i: (0, 0)),
        pl.BlockSpec((1, h), lambda i: (0, 0)),
    ]
    args = (x, tt, pc, dty, w, b)
    if prev is None:
        body = functools.partial(_ln_body, None)
        in_specs = specs
        aliases = {}
    else:
        body = _ln_body
        in_specs = [pl.BlockSpec(memory_space=pl.ANY)] + specs
        args = (prev,) + args
        aliases = {0: 0}
    return pl.pallas_call(
        body,
        grid=(nblk,),
        in_specs=in_specs,
        out_specs=pl.BlockSpec((bb, l, h), lambda i, _c=c, _n=nblk: (_c * _n + i, 0, 0)),
        out_shape=jax.ShapeDtypeStruct((bsz, l, h), jnp.float32),
        input_output_aliases=aliases,
    )(*args)


def kernel(input_ids, token_type_ids, word_emb, pos_emb, type_emb, ln_w, ln_b):
    bsz, l = input_ids.shape
    h = word_emb.shape[1]
    bc = bsz // _NCH
    ids = input_ids.reshape(_NCH, _NW, -1, _GC).astype(jnp.int32)
    tt = token_type_ids.reshape(_NCH, bc, l).astype(jnp.float32)
    pc = pos_emb[:l] + type_emb[0]
    dty = (type_emb[1] - type_emb[0]).reshape(1, h)
    w = ln_w.reshape(1, h)
    b = ln_b.reshape(1, h)
    out = None
    for c in range(_NCH):
        words = _sc_gather(word_emb, ids[c])
        out = _tc_ln(out, c, bsz, words.reshape(bc, l, h), tt[c], pc, dty, w, b)
    return out
